# baseline (speedup 1.0000x reference)
import jax
import jax.numpy as jnp
from jax import lax
from jax.experimental import pallas as pl
from jax.experimental.pallas import tpu as pltpu
from jax.experimental.pallas import tpu_sc as plsc

_N_ROWS = 4096
_N_FIELDS = 26
_FIELD_SIZE = 1000
_ROW_WORDS = _N_FIELDS * _FIELD_SIZE
_NUM_CORES = 2
_NUM_SUBCORES = 16
_NUM_WORKERS = _NUM_CORES * _NUM_SUBCORES
_ROWS_PER_W = _N_ROWS // _NUM_WORKERS
_CODES_PER_W = _ROWS_PER_W * _N_FIELDS
_STRIPE = 8
_NQ = 4
_CHUNK_W = 6400
_TAIL_START = _NQ * _CHUNK_W
_TAIL_W = _ROW_WORDS - _TAIL_START
_N_STRIPES_W = _ROWS_PER_W // _STRIPE


def _sc_body(inp_ref, out_ref, b0, b1, t0, t1, codes, s0, s1, ts0, ts1):
    bufs = (b0, b1)
    sems = (s0, s1)
    tbufs = (t0, t1)
    tsems = (ts0, ts1)
    wid = lax.axis_index("s") * _NUM_CORES + lax.axis_index("c")
    pltpu.sync_copy(inp_ref.at[pl.ds(wid * _CODES_PER_W, _CODES_PER_W)], codes)

    zeros = jnp.zeros((16,), jnp.float32)
    ones = jnp.ones((16,), jnp.float32)
    iota = lax.iota(jnp.int32, 16)
    mask_hi = iota >= 6

    def zero_fill(bb, width):
        def zero_row(s, carry):
            def zero_body(i, carry2):
                bb[s, pl.ds(i * 16, 16)] = zeros
                return carry2

            return lax.fori_loop(0, width // 16, zero_body, carry)

        lax.fori_loop(0, _STRIPE, zero_row, 0)

    for b in range(2):
        zero_fill(bufs[b], _CHUNK_W)
        zero_fill(tbufs[b], _TAIL_W)

    row_base = wid * _ROWS_PER_W

    def write_marks(bb, stripe_l, cstart, width, val):
        for s in range(_STRIPE):
            rl = stripe_l * _STRIPE + s
            c0 = codes[pl.ds(rl * _N_FIELDS, 16)].astype(jnp.int32)
            c1 = codes[pl.ds(rl * _N_FIELDS + 10, 16)].astype(jnp.int32)
            pos0 = iota * _FIELD_SIZE + c0 - cstart
            pos1 = (iota + 10) * _FIELD_SIZE + c1 - cstart
            m0 = (pos0 >= 0) & (pos0 < width)
            m1 = mask_hi & (pos1 >= 0) & (pos1 < width)
            svec = jnp.full((16,), s, jnp.int32)
            plsc.store_scatter(bb, [svec, pos0], val, mask=m0)
            plsc.store_scatter(bb, [svec, pos1], val, mask=m1)

    def chunk_dst(ci):
        stripe_l = ci // _NQ
        q = ci % _NQ
        row0 = row_base + stripe_l * _STRIPE
        return out_ref.at[
            pl.ds(row0, _STRIPE), pl.ds(q * _CHUNK_W, _CHUNK_W)
        ]

    def tail_dst(stripe_l):
        row0 = row_base + stripe_l * _STRIPE
        return out_ref.at[pl.ds(row0, _STRIPE), pl.ds(_TAIL_START, _TAIL_W)]

    n_chunks = _N_STRIPES_W * _NQ

    def group_body(g, carry):
        for b in range(2):
            bb = bufs[b]
            ci = g * 2 + b

            @pl.when(g > 0)
            def _(bb=bb, ci=ci, b=b):
                pltpu.make_async_copy(bb, chunk_dst(ci - 2), sems[b]).wait()
                oci = ci - 2
                write_marks(bb, oci // _NQ, (oci % _NQ) * _CHUNK_W,
                            _CHUNK_W, zeros)

            write_marks(bb, ci // _NQ, (ci % _NQ) * _CHUNK_W, _CHUNK_W, ones)
            pltpu.async_copy(bb, chunk_dst(ci), sems[b])
        return carry

    lax.fori_loop(0, n_chunks // 2, group_body, 0)

    def tail_body(g, carry):
        for b in range(2):
            tb = tbufs[b]
            stripe_l = g * 2 + b

            @pl.when(g > 0)
            def _(tb=tb, stripe_l=stripe_l, b=b):
                pltpu.make_async_copy(
                    tb, tail_dst(stripe_l - 2), tsems[b]
                ).wait()
                write_marks(tb, stripe_l - 2, _TAIL_START, _TAIL_W, zeros)

            write_marks(tb, stripe_l, _TAIL_START, _TAIL_W, ones)
            pltpu.async_copy(tb, tail_dst(stripe_l), tsems[b])
        return carry

    lax.fori_loop(0, _N_STRIPES_W // 2, tail_body, 0)

    for b in range(2):
        pltpu.make_async_copy(
            bufs[b], chunk_dst(n_chunks - 2 + b), sems[b]
        ).wait()
        pltpu.make_async_copy(
            tbufs[b], tail_dst(_N_STRIPES_W - 2 + b), tsems[b]
        ).wait()


def kernel(input):
    n = input.shape[0]
    flat_in = input.reshape(-1)
    mesh = plsc.VectorSubcoreMesh(
        core_axis_name="c", subcore_axis_name="s"
    )
    out = pl.kernel(
        _sc_body,
        out_type=jax.ShapeDtypeStruct((n, _ROW_WORDS), jnp.float32),
        mesh=mesh,
        compiler_params=pltpu.CompilerParams(
            needs_layout_passes=False, use_tc_tiling_on_sc=True
        ),
        scratch_types=[
            pltpu.VMEM((_STRIPE, _CHUNK_W), jnp.float32),
            pltpu.VMEM((_STRIPE, _CHUNK_W), jnp.float32),
            pltpu.VMEM((_STRIPE, _TAIL_W), jnp.float32),
            pltpu.VMEM((_STRIPE, _TAIL_W), jnp.float32),
            pltpu.VMEM((_CODES_PER_W,), jnp.float32),
            pltpu.SemaphoreType.DMA,
            pltpu.SemaphoreType.DMA,
            pltpu.SemaphoreType.DMA,
            pltpu.SemaphoreType.DMA,
        ],
    )(flat_in)
    return out
